# baseline (speedup 1.0000x reference)
import functools

import jax
import jax.numpy as jnp
from jax import lax
from jax.experimental import pallas as pl
from jax.experimental.pallas import tpu as pltpu
from jax.experimental.pallas import tpu_sc as plsc

N = 10000
E = 160000
NF = 128
EF = 16
H = 128
NED = 16

NC = 2
NS = 16
NW = NC * NS
EPW = E // NW
CH = 128
NCH = 40
EPWP = NCH * CH
RPT = N // NS

EB = 3200
EG = E // EB
EP = E // 8
NP = N // 8


def _pack8(m):
    k = m.shape[0] // 8
    return jnp.concatenate([m[k * s:k * (s + 1), :] for s in range(8)], axis=1)


def _unpack8(p):
    return jnp.concatenate([p[:, NED * s:NED * (s + 1)] for s in range(8)],
                           axis=0)


def _h0_body(x3_ref, wn_ref, bn_ref, o_ref):
    x3 = x3_ref[...]
    o_ref[...] = jnp.concatenate(
        [jnp.maximum(x3[:, sub, :] @ wn_ref[...] + bn_ref[...], 0.0)
         for sub in range(8)], axis=1)


def _msg_body(eap, xi, xj, w1bd, b1t, w2, b2, r, s2, o_msg):
    hp = jnp.maximum(eap[...] @ w1bd[...] + b1t[...], 0.0)
    hid = jnp.concatenate([hp[:, H * sub:H * (sub + 1)] for sub in range(8)],
                          axis=0)
    w = jnp.dot(hid.astype(jnp.bfloat16), w2[...],
                preferred_element_type=jnp.float32) + b2[...]
    nf = jnp.concatenate([_unpack8(xi[...]), _unpack8(xj[...])], axis=1)
    nfx = jnp.dot(nf.astype(jnp.bfloat16), r[...],
                  preferred_element_type=jnp.float32)
    t = nfx * w
    t = t[:, :256] + t[:, 256:]
    t = t[:, :128] + t[:, 128:]
    msg = jnp.dot(t.astype(jnp.bfloat16), s2[...],
                  preferred_element_type=jnp.float32)
    o_msg[...] = _pack8(msg)


def _msg_emb_body(eap, xi, xj, w1bd, b1t, w2, b2, r, s2, webd, be8, g, gt,
                  o_msg, o_em, o_lp):
    _msg_body(eap, xi, xj, w1bd, b1t, w2, b2, r, s2, o_msg)
    emp = jnp.maximum(eap[...] @ webd[...] + be8[...], 0.0)
    gsum = jnp.exp(emp) @ g[...]
    lsex = jnp.log(gsum) @ gt[...]
    o_em[...] = emp
    o_lp[...] = emp - lsex


def _upd_body(part, h, rootbd, bias8, o_ref):
    o_ref[...] = part[0] + part[1] + h[...] @ rootbd[...] + bias8[...]


def _full(shape):
    return pl.BlockSpec(shape, lambda i: tuple(0 for _ in shape))


_EBLK = pl.BlockSpec((EB, NED), lambda i: (i, 0))
_PBLK = pl.BlockSpec((EB // 8, 128), lambda i: (i, 0))

_msg_specs = [
    _PBLK,
    _PBLK,
    _PBLK,
    _full((128, 8 * H)),
    _full((1, 8 * H)),
    _full((H, 2 * NED * NED)),
    _full((1, 2 * NED * NED)),
    _full((2 * NED, 2 * NED * NED)),
    _full((128, NED)),
]

_msg_call = pl.pallas_call(
    _msg_body,
    grid=(EG,),
    in_specs=_msg_specs,
    out_specs=_PBLK,
    out_shape=jax.ShapeDtypeStruct((EP, 128), jnp.float32),
)

_msg_emb_call = pl.pallas_call(
    _msg_emb_body,
    grid=(EG,),
    in_specs=_msg_specs + [_full((128, 128)), _full((1, 128)),
                           _full((128, 8)), _full((8, 128))],
    out_specs=(_PBLK, _PBLK, _PBLK),
    out_shape=(jax.ShapeDtypeStruct((EP, 128), jnp.float32),
               jax.ShapeDtypeStruct((EP, 128), jnp.float32),
               jax.ShapeDtypeStruct((EP, 128), jnp.float32)),
)

_h0_call = pl.pallas_call(
    _h0_body,
    out_shape=jax.ShapeDtypeStruct((NP, 128), jnp.float32),
)

_upd_call = pl.pallas_call(
    _upd_body,
    out_shape=jax.ShapeDtypeStruct((NP, 128), jnp.float32),
)


def _gather_core(h_hbm, dst3, src3, xi_hbm, xj_hbm, idx_v, rows_v, sem, wid):
    for idx_hbm, out_hbm in ((dst3, xi_hbm), (src3, xj_hbm)):
        pltpu.sync_copy(idx_hbm.at[wid], idx_v)

        def grp(g, _):
            descs = []
            for k in range(8):
                j = g * 8 + k
                descs.append(pltpu.async_copy(
                    h_hbm.at[idx_v.at[j]],
                    rows_v.at[pl.ds(j * CH, CH)], sem))
            for d in descs:
                d.wait()
            return 0

        lax.fori_loop(0, NCH // 8, grp, 0)
        pltpu.sync_copy(rows_v.at[pl.ds(0, EPW)], out_hbm.at[wid])


def _gather_body(h_hbm, dst3, src3, xi_hbm, xj_hbm, idx_v, rows_v, sem):
    wid = lax.axis_index("s") * NC + lax.axis_index("c")
    _gather_core(h_hbm, dst3, src3, xi_hbm, xj_hbm, idx_v, rows_v, sem, wid)


def _gather_ea_body(h_hbm, dst3, src3, ea_hbm, xi_hbm, xj_hbm, eap_hbm,
                    idx_v, rows_v, sem):
    wid = lax.axis_index("s") * NC + lax.axis_index("c")
    pltpu.sync_copy(ea_hbm.at[pl.ds(wid * EPW, EPW)], eap_hbm.at[wid])
    _gather_core(h_hbm, dst3, src3, xi_hbm, xj_hbm, idx_v, rows_v, sem, wid)


def _scatter_core(msg_hbm, dst3, out_hbm, idx_v, rows_v, zero_v, acc_sh,
                  cid, sid, wid):
    def zr(i, _):
        zero_v[i, :] = jnp.zeros((NED,), jnp.float32)
        return 0

    lax.fori_loop(0, RPT, zr, 0)
    pltpu.sync_copy(zero_v, acc_sh.at[pl.ds(sid * RPT, RPT)])

    def zr2(i, _):
        rows_v[EPW + i, :] = jnp.zeros((NED,), jnp.float32)
        return 0

    lax.fori_loop(0, EPWP - EPW, zr2, 0)
    plsc.subcore_barrier()

    pltpu.sync_copy(dst3.at[wid], idx_v)
    pltpu.sync_copy(msg_hbm.at[wid], rows_v.at[pl.ds(0, EPW)])

    def body(j, _):
        pltpu.sync_copy(rows_v.at[pl.ds(j * CH, CH)],
                        acc_sh.at[idx_v.at[j]], add=True)
        return 0

    lax.fori_loop(0, NCH, body, 0)
    plsc.subcore_barrier()
    pltpu.sync_copy(acc_sh.at[pl.ds(sid * RPT, RPT)],
                    out_hbm.at[cid, pl.ds(sid * RPT, RPT)])


def _scatter_body(msg_hbm, dst3, out_hbm, idx_v, rows_v, zero_v, acc_sh):
    cid = lax.axis_index("c")
    sid = lax.axis_index("s")
    wid = sid * NC + cid
    _scatter_core(msg_hbm, dst3, out_hbm, idx_v, rows_v, zero_v, acc_sh,
                  cid, sid, wid)


def _scatter_emb_body(msg_hbm, dst3, emp_hbm, lpp_hbm, out_hbm, em_hbm,
                      lp_hbm, idx_v, rows_v, zero_v, acc_sh):
    cid = lax.axis_index("c")
    sid = lax.axis_index("s")
    wid = sid * NC + cid
    pltpu.sync_copy(emp_hbm.at[wid], em_hbm.at[pl.ds(wid * EPW, EPW)])
    pltpu.sync_copy(lpp_hbm.at[wid], lp_hbm.at[pl.ds(wid * EPW, EPW)])
    _scatter_core(msg_hbm, dst3, out_hbm, idx_v, rows_v, zero_v, acc_sh,
                  cid, sid, wid)


@functools.cache
def _sc_kernels():
    mesh = plsc.VectorSubcoreMesh(core_axis_name="c", subcore_axis_name="s",
                                  num_cores=NC, num_subcores=NS)
    params = pltpu.CompilerParams(use_tc_tiling_on_sc=False)
    g_scratch = [
        pltpu.VMEM((NCH, CH), jnp.int32),
        pltpu.VMEM((EPWP, NED), jnp.float32),
        pltpu.SemaphoreType.DMA,
    ]
    s_scratch = [
        pltpu.VMEM((NCH, CH), jnp.int32),
        pltpu.VMEM((EPWP, NED), jnp.float32),
        pltpu.VMEM((RPT, NED), jnp.float32),
        pltpu.VMEM_SHARED((N, NED), jnp.float32),
    ]
    epw16 = jax.ShapeDtypeStruct((NW, EPW, NED), jnp.float32)
    e16 = jax.ShapeDtypeStruct((E, NED), jnp.float32)
    gather = pl.kernel(
        _gather_body, compiler_params=params, mesh=mesh,
        out_type=(epw16, epw16), scratch_types=g_scratch)
    gather_ea = pl.kernel(
        _gather_ea_body, compiler_params=params, mesh=mesh,
        out_type=(epw16, epw16, epw16), scratch_types=g_scratch)
    part_t = jax.ShapeDtypeStruct((NC, N, NED), jnp.float32)
    scatter = pl.kernel(
        _scatter_body, compiler_params=params, mesh=mesh,
        out_type=part_t, scratch_types=s_scratch)
    scatter_emb = pl.kernel(
        _scatter_emb_body, compiler_params=params, mesh=mesh,
        out_type=(part_t, e16, e16), scratch_types=s_scratch)
    return gather, gather_ea, scatter, scatter_emb


def kernel(x, edge_index, edge_attr, Wn, bn, We, be, W1, b1, W2, b2,
           root1, bias1, root2, bias2):
    _gather, _gather_ea, _scatter, _scatter_emb = _sc_kernels()

    def sc_idx(a):
        padded = jnp.pad(a.reshape(NW, EPW), ((0, 0), (0, EPWP - EPW)))
        return padded.reshape(NW, NCH, CH)

    src3 = sc_idx(edge_index[0])
    dst3 = sc_idx(edge_index[1])

    r = jnp.kron(jnp.eye(2 * NED, dtype=jnp.bfloat16),
                 jnp.ones((1, NED), jnp.bfloat16))
    s2 = jnp.tile(jnp.eye(NED, dtype=jnp.bfloat16), (8, 1))
    g = jnp.kron(jnp.eye(8, dtype=jnp.float32),
                 jnp.ones((NED, 1), jnp.float32))
    gt = jnp.kron(jnp.eye(8, dtype=jnp.float32),
                  jnp.ones((1, NED), jnp.float32))

    eye8 = jnp.eye(8, dtype=jnp.float32)
    bn_ = bn.reshape(1, NED)
    b2_ = b2.reshape(1, 2 * NED * NED)
    w2bf = W2.astype(jnp.bfloat16)
    w1bd = jnp.kron(eye8, W1)
    b1t = jnp.tile(b1, 8).reshape(1, 8 * H)
    webd = jnp.kron(eye8, We)
    be8 = jnp.tile(be, 8).reshape(1, 128)
    root1bd = jnp.kron(eye8, root1)
    root2bd = jnp.kron(eye8, root2)
    bias1_8 = jnp.tile(bias1, 8).reshape(1, 128)
    bias2_8 = jnp.tile(bias2, 8).reshape(1, 128)

    x3 = x.reshape(NP, 8, NF)
    h0p = _h0_call(x3, Wn, bn_)

    xi1, xj1, eapw = _gather_ea(h0p.reshape(N, NED), dst3, src3, edge_attr)
    eap = eapw.reshape(EP, 128)
    msg1, emp, lpp = _msg_emb_call(eap, xi1.reshape(EP, 128),
                                   xj1.reshape(EP, 128), w1bd, b1t, w2bf,
                                   b2_, r, s2, webd, be8, g, gt)
    part1, em, lp = _scatter_emb(msg1.reshape(NW, EPW, NED), dst3,
                                 emp.reshape(NW, EPW, NED),
                                 lpp.reshape(NW, EPW, NED))
    h1p = _upd_call(part1.reshape(NC, NP, 128), h0p, root1bd, bias1_8)

    xi2, xj2 = _gather(h1p.reshape(N, NED), dst3, src3)
    msg2 = _msg_call(eap, xi2.reshape(EP, 128), xj2.reshape(EP, 128),
                     w1bd, b1t, w2bf, b2_, r, s2)
    part2 = _scatter(msg2.reshape(NW, EPW, NED), dst3)
    h2p = _upd_call(part2.reshape(NC, NP, 128), h1p, root2bd, bias2_8)

    return (h2p.reshape(N, NED), edge_index, em, lp)


# speedup vs baseline: 2.4157x; 2.4157x over previous
import functools

import jax
import jax.numpy as jnp
from jax import lax
from jax.experimental import pallas as pl
from jax.experimental.pallas import tpu as pltpu
from jax.experimental.pallas import tpu_sc as plsc

N = 10000
E = 160000
NF = 128
EF = 16
H = 128
NED = 16

NC = 2
NS = 16
NW = NC * NS
EPW = E // NW
CH = 64
NCH = 80
EPWP = NCH * CH
RPT = N // NS

EB = 3200
EG = E // EB
EP = E // 8
NP = N // 8


def _pack8(m):
    k = m.shape[0] // 8
    return jnp.concatenate([m[k * s:k * (s + 1), :] for s in range(8)], axis=1)


def _unpack8(p):
    return jnp.concatenate([p[:, NED * s:NED * (s + 1)] for s in range(8)],
                           axis=0)


def _h0_body(x3_ref, wn_ref, bn_ref, o_ref):
    x3 = x3_ref[...]
    o_ref[...] = jnp.concatenate(
        [jnp.maximum(x3[:, sub, :] @ wn_ref[...] + bn_ref[...], 0.0)
         for sub in range(8)], axis=1)


def _msg_body(eap, xi, xj, w1bd, b1t, w2, b2, r, s2, o_msg):
    hp = jnp.maximum(eap[...] @ w1bd[...] + b1t[...], 0.0)
    hid = jnp.concatenate([hp[:, H * sub:H * (sub + 1)] for sub in range(8)],
                          axis=0)
    w = jnp.dot(hid.astype(jnp.bfloat16), w2[...],
                preferred_element_type=jnp.float32) + b2[...]
    nf = jnp.concatenate([_unpack8(xi[...]), _unpack8(xj[...])], axis=1)
    nfx = jnp.dot(nf.astype(jnp.bfloat16), r[...],
                  preferred_element_type=jnp.float32)
    t = nfx * w
    t = t[:, :256] + t[:, 256:]
    t = t[:, :128] + t[:, 128:]
    msg = jnp.dot(t.astype(jnp.bfloat16), s2[...],
                  preferred_element_type=jnp.float32)
    o_msg[...] = _pack8(msg)


def _msg_emb_body(eap, xi, xj, w1bd, b1t, w2, b2, r, s2, webd, be8, g, gt,
                  o_msg, o_em, o_lp):
    _msg_body(eap, xi, xj, w1bd, b1t, w2, b2, r, s2, o_msg)
    emp = jnp.maximum(eap[...] @ webd[...] + be8[...], 0.0)
    gsum = jnp.exp(emp) @ g[...]
    lsex = jnp.log(gsum) @ gt[...]
    o_em[...] = emp
    o_lp[...] = emp - lsex


def _upd_body(part, h, rootbd, bias8, o_ref):
    o_ref[...] = part[0] + part[1] + h[...] @ rootbd[...] + bias8[...]


def _full(shape):
    return pl.BlockSpec(shape, lambda i: tuple(0 for _ in shape))


_EBLK = pl.BlockSpec((EB, NED), lambda i: (i, 0))
_PBLK = pl.BlockSpec((EB // 8, 128), lambda i: (i, 0))

_msg_specs = [
    _PBLK,
    _PBLK,
    _PBLK,
    _full((128, 8 * H)),
    _full((1, 8 * H)),
    _full((H, 2 * NED * NED)),
    _full((1, 2 * NED * NED)),
    _full((2 * NED, 2 * NED * NED)),
    _full((128, NED)),
]

_msg_call = pl.pallas_call(
    _msg_body,
    grid=(EG,),
    in_specs=_msg_specs,
    out_specs=_PBLK,
    out_shape=jax.ShapeDtypeStruct((EP, 128), jnp.float32),
)

_msg_emb_call = pl.pallas_call(
    _msg_emb_body,
    grid=(EG,),
    in_specs=_msg_specs + [_full((128, 128)), _full((1, 128)),
                           _full((128, 8)), _full((8, 128))],
    out_specs=(_PBLK, _PBLK, _PBLK),
    out_shape=(jax.ShapeDtypeStruct((EP, 128), jnp.float32),
               jax.ShapeDtypeStruct((EP, 128), jnp.float32),
               jax.ShapeDtypeStruct((EP, 128), jnp.float32)),
)

_h0_call = pl.pallas_call(
    _h0_body,
    out_shape=jax.ShapeDtypeStruct((NP, 128), jnp.float32),
)

_upd_call = pl.pallas_call(
    _upd_body,
    out_shape=jax.ShapeDtypeStruct((NP, 128), jnp.float32),
)


def _gather_core(h_hbm, dst3, src3, xi_hbm, xj_hbm, idx_v, rows_v, sem, wid):
    for idx_hbm, out_hbm in ((dst3, xi_hbm), (src3, xj_hbm)):
        pltpu.sync_copy(idx_hbm.at[wid], idx_v)

        def grp(g, _):
            descs = []
            for k in range(8):
                j = g * 8 + k
                descs.append(pltpu.async_copy(
                    h_hbm.at[idx_v.at[j]],
                    rows_v.at[pl.ds(j * CH, CH)], sem))
            for d in descs:
                d.wait()
            return 0

        lax.fori_loop(0, NCH // 8, grp, 0)
        pltpu.sync_copy(rows_v.at[pl.ds(0, EPW)], out_hbm.at[wid])


def _gather_body(h_hbm, dst3, src3, xi_hbm, xj_hbm, idx_v, rows_v, sem):
    wid = lax.axis_index("s") * NC + lax.axis_index("c")
    _gather_core(h_hbm, dst3, src3, xi_hbm, xj_hbm, idx_v, rows_v, sem, wid)


def _gather_ea_body(h_hbm, dst3, src3, ea_hbm, xi_hbm, xj_hbm, eap_hbm,
                    idx_v, rows_v, sem):
    wid = lax.axis_index("s") * NC + lax.axis_index("c")
    pltpu.sync_copy(ea_hbm.at[pl.ds(wid * EPW, EPW)], eap_hbm.at[wid])
    _gather_core(h_hbm, dst3, src3, xi_hbm, xj_hbm, idx_v, rows_v, sem, wid)


def _scatter_core(msg_hbm, dst3, out_hbm, idx_v, rows_v, zero_v, acc_sh,
                  cid, sid, wid):
    def zr(i, _):
        zero_v[i, :] = jnp.zeros((NED,), jnp.float32)
        return 0

    lax.fori_loop(0, RPT, zr, 0)
    pltpu.sync_copy(zero_v, acc_sh.at[pl.ds(sid * RPT, RPT)])

    def zr2(i, _):
        rows_v[EPW + i, :] = jnp.zeros((NED,), jnp.float32)
        return 0

    lax.fori_loop(0, EPWP - EPW, zr2, 0)
    plsc.subcore_barrier()

    pltpu.sync_copy(dst3.at[wid], idx_v)
    pltpu.sync_copy(msg_hbm.at[wid], rows_v.at[pl.ds(0, EPW)])

    def body(j, _):
        pltpu.sync_copy(rows_v.at[pl.ds(j * CH, CH)],
                        acc_sh.at[idx_v.at[j]], add=True)
        return 0

    lax.fori_loop(0, NCH, body, 0)
    plsc.subcore_barrier()
    pltpu.sync_copy(acc_sh.at[pl.ds(sid * RPT, RPT)],
                    out_hbm.at[cid, pl.ds(sid * RPT, RPT)])


def _scatter_body(msg_hbm, dst3, out_hbm, idx_v, rows_v, zero_v, acc_sh):
    cid = lax.axis_index("c")
    sid = lax.axis_index("s")
    wid = sid * NC + cid
    _scatter_core(msg_hbm, dst3, out_hbm, idx_v, rows_v, zero_v, acc_sh,
                  cid, sid, wid)


def _scatter_emb_body(msg_hbm, dst3, emp_hbm, lpp_hbm, out_hbm, em_hbm,
                      lp_hbm, idx_v, rows_v, zero_v, acc_sh):
    cid = lax.axis_index("c")
    sid = lax.axis_index("s")
    wid = sid * NC + cid
    pltpu.sync_copy(emp_hbm.at[wid], em_hbm.at[pl.ds(wid * EPW, EPW)])
    pltpu.sync_copy(lpp_hbm.at[wid], lp_hbm.at[pl.ds(wid * EPW, EPW)])
    _scatter_core(msg_hbm, dst3, out_hbm, idx_v, rows_v, zero_v, acc_sh,
                  cid, sid, wid)


@functools.cache
def _sc_kernels():
    mesh = plsc.VectorSubcoreMesh(core_axis_name="c", subcore_axis_name="s",
                                  num_cores=NC, num_subcores=NS)
    params = pltpu.CompilerParams(use_tc_tiling_on_sc=False)
    g_scratch = [
        pltpu.VMEM((NCH, CH), jnp.int32),
        pltpu.VMEM((EPWP, NED), jnp.float32),
        pltpu.SemaphoreType.DMA,
    ]
    s_scratch = [
        pltpu.VMEM((NCH, CH), jnp.int32),
        pltpu.VMEM((EPWP, NED), jnp.float32),
        pltpu.VMEM((RPT, NED), jnp.float32),
        pltpu.VMEM_SHARED((N, NED), jnp.float32),
    ]
    epw16 = jax.ShapeDtypeStruct((NW, EPW, NED), jnp.float32)
    e16 = jax.ShapeDtypeStruct((E, NED), jnp.float32)
    gather = pl.kernel(
        _gather_body, compiler_params=params, mesh=mesh,
        out_type=(epw16, epw16), scratch_types=g_scratch)
    gather_ea = pl.kernel(
        _gather_ea_body, compiler_params=params, mesh=mesh,
        out_type=(epw16, epw16, epw16), scratch_types=g_scratch)
    part_t = jax.ShapeDtypeStruct((NC, N, NED), jnp.float32)
    scatter = pl.kernel(
        _scatter_body, compiler_params=params, mesh=mesh,
        out_type=part_t, scratch_types=s_scratch)
    scatter_emb = pl.kernel(
        _scatter_emb_body, compiler_params=params, mesh=mesh,
        out_type=(part_t, e16, e16), scratch_types=s_scratch)
    return gather, gather_ea, scatter, scatter_emb


def kernel(x, edge_index, edge_attr, Wn, bn, We, be, W1, b1, W2, b2,
           root1, bias1, root2, bias2):
    _gather, _gather_ea, _scatter, _scatter_emb = _sc_kernels()

    def sc_idx(a):
        padded = jnp.pad(a.reshape(NW, EPW), ((0, 0), (0, EPWP - EPW)))
        return padded.reshape(NW, NCH, CH)

    src3 = sc_idx(edge_index[0])
    dst3 = sc_idx(edge_index[1])

    r = jnp.kron(jnp.eye(2 * NED, dtype=jnp.bfloat16),
                 jnp.ones((1, NED), jnp.bfloat16))
    s2 = jnp.tile(jnp.eye(NED, dtype=jnp.bfloat16), (8, 1))
    g = jnp.kron(jnp.eye(8, dtype=jnp.float32),
                 jnp.ones((NED, 1), jnp.float32))
    gt = jnp.kron(jnp.eye(8, dtype=jnp.float32),
                  jnp.ones((1, NED), jnp.float32))

    eye8 = jnp.eye(8, dtype=jnp.float32)
    bn_ = bn.reshape(1, NED)
    b2_ = b2.reshape(1, 2 * NED * NED)
    w2bf = W2.astype(jnp.bfloat16)
    w1bd = jnp.kron(eye8, W1)
    b1t = jnp.tile(b1, 8).reshape(1, 8 * H)
    webd = jnp.kron(eye8, We)
    be8 = jnp.tile(be, 8).reshape(1, 128)
    root1bd = jnp.kron(eye8, root1)
    root2bd = jnp.kron(eye8, root2)
    bias1_8 = jnp.tile(bias1, 8).reshape(1, 128)
    bias2_8 = jnp.tile(bias2, 8).reshape(1, 128)

    x3 = x.reshape(NP, 8, NF)
    eap = edge_attr.reshape(EP, 128)
    h0p = _h0_call(x3, Wn, bn_)

    xi1, xj1 = _gather(h0p.reshape(N, NED), dst3, src3)
    msg1, emp, lpp = _msg_emb_call(eap, xi1.reshape(EP, 128),
                                   xj1.reshape(EP, 128), w1bd, b1t, w2bf,
                                   b2_, r, s2, webd, be8, g, gt)
    part1 = _scatter(msg1.reshape(NW, EPW, NED), dst3)
    h1p = _upd_call(part1.reshape(NC, NP, 128), h0p, root1bd, bias1_8)

    xi2, xj2 = _gather(h1p.reshape(N, NED), dst3, src3)
    msg2 = _msg_call(eap, xi2.reshape(EP, 128), xj2.reshape(EP, 128),
                     w1bd, b1t, w2bf, b2_, r, s2)
    part2 = _scatter(msg2.reshape(NW, EPW, NED), dst3)
    h2p = _upd_call(part2.reshape(NC, NP, 128), h1p, root2bd, bias2_8)

    return (h2p.reshape(N, NED), edge_index, emp.reshape(E, NED),
            lpp.reshape(E, NED))


# speedup vs baseline: 2.4592x; 1.0180x over previous
import functools

import jax
import jax.numpy as jnp
from jax import lax
from jax.experimental import pallas as pl
from jax.experimental.pallas import tpu as pltpu
from jax.experimental.pallas import tpu_sc as plsc

N = 10000
E = 160000
NF = 128
EF = 16
H = 128
NED = 16

NC = 2
NS = 16
NW = NC * NS
EPW = E // NW
CH = 128
NCH = 40
EPWP = NCH * CH
RPT = N // NS

EB = 3200
EG = E // EB
EP = E // 8
NP = N // 8


def _pack8(m):
    k = m.shape[0] // 8
    return jnp.concatenate([m[k * s:k * (s + 1), :] for s in range(8)], axis=1)


def _unpack8(p):
    return jnp.concatenate([p[:, NED * s:NED * (s + 1)] for s in range(8)],
                           axis=0)


def _h0_body(x3_ref, wn_ref, bn_ref, o_ref):
    x3 = x3_ref[...]
    o_ref[...] = jnp.concatenate(
        [jnp.maximum(x3[:, sub, :] @ wn_ref[...] + bn_ref[...], 0.0)
         for sub in range(8)], axis=1)


def _msg_body(eap, xi, xj, w1bd, b1t, w2, b2, r, s2, o_msg):
    hp = jnp.maximum(eap[...] @ w1bd[...] + b1t[...], 0.0)
    hid = jnp.concatenate([hp[:, H * sub:H * (sub + 1)] for sub in range(8)],
                          axis=0)
    w = jnp.dot(hid.astype(jnp.bfloat16), w2[...],
                preferred_element_type=jnp.float32) + b2[...]
    nf = jnp.concatenate([_unpack8(xi[...]), _unpack8(xj[...])], axis=1)
    nfx = jnp.dot(nf.astype(jnp.bfloat16), r[...],
                  preferred_element_type=jnp.float32)
    t = nfx * w
    t = t[:, :256] + t[:, 256:]
    t = t[:, :128] + t[:, 128:]
    msg = jnp.dot(t.astype(jnp.bfloat16), s2[...],
                  preferred_element_type=jnp.float32)
    o_msg[...] = _pack8(msg)


def _msg_emb_body(eap, xi, xj, w1bd, b1t, w2, b2, r, s2, webd, be8, g, gt,
                  o_msg, o_em, o_lp):
    _msg_body(eap, xi, xj, w1bd, b1t, w2, b2, r, s2, o_msg)
    emp = jnp.maximum(eap[...] @ webd[...] + be8[...], 0.0)
    gsum = jnp.exp(emp) @ g[...]
    lsex = jnp.log(gsum) @ gt[...]
    o_em[...] = emp
    o_lp[...] = emp - lsex


def _upd_body(part, h, rootbd, bias8, o_ref):
    o_ref[...] = part[0] + part[1] + h[...] @ rootbd[...] + bias8[...]


def _full(shape):
    return pl.BlockSpec(shape, lambda i: tuple(0 for _ in shape))


_EBLK = pl.BlockSpec((EB, NED), lambda i: (i, 0))
_PBLK = pl.BlockSpec((EB // 8, 128), lambda i: (i, 0))

_msg_specs = [
    _PBLK,
    _PBLK,
    _PBLK,
    _full((128, 8 * H)),
    _full((1, 8 * H)),
    _full((H, 2 * NED * NED)),
    _full((1, 2 * NED * NED)),
    _full((2 * NED, 2 * NED * NED)),
    _full((128, NED)),
]

_msg_call = pl.pallas_call(
    _msg_body,
    grid=(EG,),
    in_specs=_msg_specs,
    out_specs=_PBLK,
    out_shape=jax.ShapeDtypeStruct((EP, 128), jnp.float32),
)

_msg_emb_call = pl.pallas_call(
    _msg_emb_body,
    grid=(EG,),
    in_specs=_msg_specs + [_full((128, 128)), _full((1, 128)),
                           _full((128, 8)), _full((8, 128))],
    out_specs=(_PBLK, _PBLK, _PBLK),
    out_shape=(jax.ShapeDtypeStruct((EP, 128), jnp.float32),
               jax.ShapeDtypeStruct((EP, 128), jnp.float32),
               jax.ShapeDtypeStruct((EP, 128), jnp.float32)),
)

_h0_call = pl.pallas_call(
    _h0_body,
    out_shape=jax.ShapeDtypeStruct((NP, 128), jnp.float32),
)

_upd_call = pl.pallas_call(
    _upd_body,
    out_shape=jax.ShapeDtypeStruct((NP, 128), jnp.float32),
)


GRP = 8 * CH
LAST = EPW - (NCH // 8 - 1) * GRP


def _gather_core(h_hbm, dst3, src3, xi_hbm, xj_hbm, idx_a, idx_b,
                 rows_a, rows_b, sem_a, sem_b, wid):
    pltpu.sync_copy(dst3.at[wid], idx_a)
    pltpu.sync_copy(src3.at[wid], idx_b)

    def fire_drain_write(g, rows):
        descs = []
        for k in range(8):
            j = g * 8 + k
            descs.append(pltpu.async_copy(
                h_hbm.at[idx_a.at[j]], rows_a.at[pl.ds(k * CH, CH)], sem_a))
            descs.append(pltpu.async_copy(
                h_hbm.at[idx_b.at[j]], rows_b.at[pl.ds(k * CH, CH)], sem_b))
        for d in descs:
            d.wait()
        pltpu.sync_copy(rows_a.at[pl.ds(0, rows)],
                        xi_hbm.at[wid, pl.ds(g * GRP, rows)])
        pltpu.sync_copy(rows_b.at[pl.ds(0, rows)],
                        xj_hbm.at[wid, pl.ds(g * GRP, rows)])

    def grp(g, _):
        fire_drain_write(g, GRP)
        return 0

    lax.fori_loop(0, NCH // 8 - 1, grp, 0)
    fire_drain_write(NCH // 8 - 1, LAST)


def _gather_body(h_hbm, dst3, src3, xi_hbm, xj_hbm, idx_a, idx_b,
                 rows_a, rows_b, sem_a, sem_b):
    wid = lax.axis_index("s") * NC + lax.axis_index("c")
    _gather_core(h_hbm, dst3, src3, xi_hbm, xj_hbm, idx_a, idx_b,
                 rows_a, rows_b, sem_a, sem_b, wid)


def _scatter_core(msg_hbm, dst3, out_hbm, idx_v, rows_v, zero_v, acc_sh,
                  cid, sid, wid):
    def zr(i, _):
        zero_v[i, :] = jnp.zeros((NED,), jnp.float32)
        return 0

    lax.fori_loop(0, RPT, zr, 0)
    pltpu.sync_copy(zero_v, acc_sh.at[pl.ds(sid * RPT, RPT)])

    def zr2(i, _):
        rows_v[EPW + i, :] = jnp.zeros((NED,), jnp.float32)
        return 0

    lax.fori_loop(0, EPWP - EPW, zr2, 0)
    plsc.subcore_barrier()

    pltpu.sync_copy(dst3.at[wid], idx_v)
    pltpu.sync_copy(msg_hbm.at[wid], rows_v.at[pl.ds(0, EPW)])

    def body(j, _):
        pltpu.sync_copy(rows_v.at[pl.ds(j * CH, CH)],
                        acc_sh.at[idx_v.at[j]], add=True)
        return 0

    lax.fori_loop(0, NCH, body, 0)
    plsc.subcore_barrier()
    pltpu.sync_copy(acc_sh.at[pl.ds(sid * RPT, RPT)],
                    out_hbm.at[cid, pl.ds(sid * RPT, RPT)])


def _scatter_body(msg_hbm, dst3, out_hbm, idx_v, rows_v, zero_v, acc_sh):
    cid = lax.axis_index("c")
    sid = lax.axis_index("s")
    wid = sid * NC + cid
    _scatter_core(msg_hbm, dst3, out_hbm, idx_v, rows_v, zero_v, acc_sh,
                  cid, sid, wid)


@functools.cache
def _sc_kernels():
    mesh = plsc.VectorSubcoreMesh(core_axis_name="c", subcore_axis_name="s",
                                  num_cores=NC, num_subcores=NS)
    params = pltpu.CompilerParams(use_tc_tiling_on_sc=False)
    g_scratch = [
        pltpu.VMEM((NCH, CH), jnp.int32),
        pltpu.VMEM((NCH, CH), jnp.int32),
        pltpu.VMEM((GRP, NED), jnp.float32),
        pltpu.VMEM((GRP, NED), jnp.float32),
        pltpu.SemaphoreType.DMA,
        pltpu.SemaphoreType.DMA,
    ]
    s_scratch = [
        pltpu.VMEM((NCH, CH), jnp.int32),
        pltpu.VMEM((EPWP, NED), jnp.float32),
        pltpu.VMEM((RPT, NED), jnp.float32),
        pltpu.VMEM_SHARED((N, NED), jnp.float32),
    ]
    epw16 = jax.ShapeDtypeStruct((NW, EPW, NED), jnp.float32)
    gather = pl.kernel(
        _gather_body, compiler_params=params, mesh=mesh,
        out_type=(epw16, epw16), scratch_types=g_scratch)
    part_t = jax.ShapeDtypeStruct((NC, N, NED), jnp.float32)
    scatter = pl.kernel(
        _scatter_body, compiler_params=params, mesh=mesh,
        out_type=part_t, scratch_types=s_scratch)
    return gather, scatter


def kernel(x, edge_index, edge_attr, Wn, bn, We, be, W1, b1, W2, b2,
           root1, bias1, root2, bias2):
    _gather, _scatter = _sc_kernels()

    def sc_idx(a):
        padded = jnp.pad(a.reshape(NW, EPW), ((0, 0), (0, EPWP - EPW)))
        return padded.reshape(NW, NCH, CH)

    src3 = sc_idx(edge_index[0])
    dst3 = sc_idx(edge_index[1])

    r = jnp.kron(jnp.eye(2 * NED, dtype=jnp.bfloat16),
                 jnp.ones((1, NED), jnp.bfloat16))
    s2 = jnp.tile(jnp.eye(NED, dtype=jnp.bfloat16), (8, 1))
    g = jnp.kron(jnp.eye(8, dtype=jnp.float32),
                 jnp.ones((NED, 1), jnp.float32))
    gt = jnp.kron(jnp.eye(8, dtype=jnp.float32),
                  jnp.ones((1, NED), jnp.float32))

    eye8 = jnp.eye(8, dtype=jnp.float32)
    bn_ = bn.reshape(1, NED)
    b2_ = b2.reshape(1, 2 * NED * NED)
    w2bf = W2.astype(jnp.bfloat16)
    w1bd = jnp.kron(eye8, W1)
    b1t = jnp.tile(b1, 8).reshape(1, 8 * H)
    webd = jnp.kron(eye8, We)
    be8 = jnp.tile(be, 8).reshape(1, 128)
    root1bd = jnp.kron(eye8, root1)
    root2bd = jnp.kron(eye8, root2)
    bias1_8 = jnp.tile(bias1, 8).reshape(1, 128)
    bias2_8 = jnp.tile(bias2, 8).reshape(1, 128)

    x3 = x.reshape(NP, 8, NF)
    eap = edge_attr.reshape(EP, 128)
    h0p = _h0_call(x3, Wn, bn_)

    xi1, xj1 = _gather(h0p.reshape(N, NED), dst3, src3)
    msg1, emp, lpp = _msg_emb_call(eap, xi1.reshape(EP, 128),
                                   xj1.reshape(EP, 128), w1bd, b1t, w2bf,
                                   b2_, r, s2, webd, be8, g, gt)
    part1 = _scatter(msg1.reshape(NW, EPW, NED), dst3)
    h1p = _upd_call(part1.reshape(NC, NP, 128), h0p, root1bd, bias1_8)

    em = emp.reshape(E, NED)
    lp = lpp.reshape(E, NED)

    xi2, xj2 = _gather(h1p.reshape(N, NED), dst3, src3)
    msg2 = _msg_call(eap, xi2.reshape(EP, 128), xj2.reshape(EP, 128),
                     w1bd, b1t, w2bf, b2_, r, s2)
    zdep = ((em[0, 0] + lp[0, 0]) * 0.0).astype(jnp.int32)
    part2 = _scatter(msg2.reshape(NW, EPW, NED), dst3 + zdep)
    h2p = _upd_call(part2.reshape(NC, NP, 128), h1p, root2bd, bias2_8)

    return (h2p.reshape(N, NED), edge_index, em, lp)
